# baseline (speedup 1.0000x reference)
import functools

import jax
import jax.numpy as jnp
from jax.experimental import pallas as pl
from jax.experimental.pallas import tpu as pltpu

_HIDDEN = 2048
_EXPERTS = 64
_TOPK = 8
_TB = 2048


def _topk_softmax(logits):
    tb = logits.shape[0]
    iota = jax.lax.broadcasted_iota(jnp.int32, (tb, _EXPERTS), 1)
    adj = logits - iota.astype(jnp.float32) * 1e-9

    vals = []
    idxs = []
    neg_inf = jnp.float32(-jnp.inf)
    for _ in range(_TOPK):
        m = jnp.max(adj, axis=1, keepdims=True)
        cand = jnp.where(adj == m, iota, _EXPERTS)
        idx = jnp.min(cand, axis=1, keepdims=True)
        sel = iota == idx
        orig = jnp.sum(jnp.where(sel, logits, 0.0), axis=1, keepdims=True)
        vals.append(orig)
        idxs.append(idx)
        adj = jnp.where(sel, neg_inf, adj)

    vals8 = jnp.concatenate(vals, axis=1)
    idx8 = jnp.concatenate(idxs, axis=1)

    m8 = jnp.max(vals8, axis=1, keepdims=True)
    e8 = jnp.exp(vals8 - m8)
    wts8 = e8 / jnp.sum(e8, axis=1, keepdims=True)
    return idx8, wts8


def _router_body(x_ref, w_ref, logits_ref, idx_ref, wts_ref, scratch_ref):
    i = pl.program_id(0)

    logits = jnp.dot(x_ref[...], w_ref[...],
                     preferred_element_type=jnp.float32)
    logits_ref[...] = logits
    scratch_ref[i % 2] = logits

    prev = scratch_ref[(i + 1) % 2]
    idx8, wts8 = _topk_softmax(prev)
    idx_ref[...] = idx8
    wts_ref[...] = wts8


@functools.partial(jax.jit, static_argnames=())
def kernel(hidden_states, W_gate):
    b, s, h = hidden_states.shape
    n = b * s
    x = hidden_states.reshape(n, h)
    nblk = n // _TB

    grid = (nblk + 1,)
    last = nblk - 1
    logits, idx8, wts8 = pl.pallas_call(
        _router_body,
        grid=grid,
        in_specs=[
            pl.BlockSpec((_TB, h), lambda i: (jnp.minimum(i, last), 0)),
            pl.BlockSpec((h, _EXPERTS), lambda i: (0, 0)),
        ],
        out_specs=[
            pl.BlockSpec((_TB, _EXPERTS), lambda i: (jnp.minimum(i, last), 0)),
            pl.BlockSpec((_TB, _TOPK), lambda i: (jnp.maximum(i - 1, 0), 0)),
            pl.BlockSpec((_TB, _TOPK), lambda i: (jnp.maximum(i - 1, 0), 0)),
        ],
        out_shape=[
            jax.ShapeDtypeStruct((n, _EXPERTS), jnp.float32),
            jax.ShapeDtypeStruct((n, _TOPK), jnp.int32),
            jax.ShapeDtypeStruct((n, _TOPK), jnp.float32),
        ],
        scratch_shapes=[pltpu.VMEM((2, _TB, _EXPERTS), jnp.float32)],
        compiler_params=pltpu.CompilerParams(
            dimension_semantics=("arbitrary",),
        ),
    )(x, W_gate)

    return (
        logits.reshape(b, s, _EXPERTS),
        idx8.reshape(b, s, _TOPK),
        wts8.reshape(b, s, _TOPK),
    )


# speedup vs baseline: 1.2905x; 1.2905x over previous
import functools

import jax
import jax.numpy as jnp
from jax.experimental import pallas as pl
from jax.experimental.pallas import tpu as pltpu

_HIDDEN = 2048
_EXPERTS = 64
_TOPK = 8
_TB = 1024


def _topk_softmax(logits):
    tb = logits.shape[0]
    iota = jax.lax.broadcasted_iota(jnp.int32, (tb, _EXPERTS), 1)
    adj = logits - iota.astype(jnp.float32) * 1e-9

    vals = []
    idxs = []
    neg_inf = jnp.float32(-jnp.inf)
    for _ in range(_TOPK):
        m = jnp.max(adj, axis=1, keepdims=True)
        cand = jnp.where(adj == m, iota, _EXPERTS)
        idx = jnp.min(cand, axis=1, keepdims=True)
        vals.append(m)
        idxs.append(idx)
        adj = jnp.where(cand == idx, neg_inf, adj)

    idx8 = jnp.concatenate(idxs, axis=1)

    es = [jnp.ones((tb, 1), jnp.float32)]
    for k in range(1, _TOPK):
        es.append(jnp.exp(vals[k] - vals[0]))
    total = es[0]
    for k in range(1, _TOPK):
        total = total + es[k]
    r = 1.0 / total
    wts8 = jnp.concatenate([e * r for e in es], axis=1)
    return idx8, wts8


def _router_body(x_ref, w_ref, logits_ref, idx_ref, wts_ref):
    logits = jnp.dot(x_ref[...], w_ref[...], preferred_element_type=jnp.float32)
    logits_ref[...] = logits
    idx8, wts8 = _topk_softmax(logits)
    idx_ref[...] = idx8
    wts_ref[...] = wts8


@functools.partial(jax.jit, static_argnames=())
def kernel(hidden_states, W_gate):
    b, s, h = hidden_states.shape
    n = b * s
    x = hidden_states.reshape(n, h)

    grid = (n // _TB,)
    logits, idx8, wts8 = pl.pallas_call(
        _router_body,
        grid=grid,
        in_specs=[
            pl.BlockSpec((_TB, h), lambda i: (i, 0)),
            pl.BlockSpec((h, _EXPERTS), lambda i: (0, 0)),
        ],
        out_specs=[
            pl.BlockSpec((_TB, _EXPERTS), lambda i: (i, 0)),
            pl.BlockSpec((_TB, _TOPK), lambda i: (i, 0)),
            pl.BlockSpec((_TB, _TOPK), lambda i: (i, 0)),
        ],
        out_shape=[
            jax.ShapeDtypeStruct((n, _EXPERTS), jnp.float32),
            jax.ShapeDtypeStruct((n, _TOPK), jnp.int32),
            jax.ShapeDtypeStruct((n, _TOPK), jnp.float32),
        ],
        compiler_params=pltpu.CompilerParams(
            dimension_semantics=("parallel",),
        ),
    )(x, W_gate)

    return (
        logits.reshape(b, s, _EXPERTS),
        idx8.reshape(b, s, _TOPK),
        wts8.reshape(b, s, _TOPK),
    )


# speedup vs baseline: 1.5631x; 1.2112x over previous
import functools

import jax
import jax.numpy as jnp
from jax.experimental import pallas as pl
from jax.experimental.pallas import tpu as pltpu

_HIDDEN = 2048
_EXPERTS = 64
_TOPK = 8
_TB = 1024


def _topk_softmax(logits):
    tb = logits.shape[0]
    iota_f = jax.lax.broadcasted_iota(jnp.int32, (tb, _EXPERTS), 1).astype(
        jnp.float32)
    adj = logits - iota_f * 1e-9

    vals = []
    idxs = []
    neg_inf = jnp.float32(-jnp.inf)
    for _ in range(_TOPK):
        m = jnp.max(adj, axis=1, keepdims=True)
        cand = jnp.where(adj == m, iota_f, jnp.float32(_EXPERTS))
        idx = jnp.min(cand, axis=1, keepdims=True)
        vals.append(m)
        idxs.append(idx)
        adj = jnp.where(cand == idx, neg_inf, adj)

    idx8 = jnp.concatenate(idxs, axis=1).astype(jnp.int32)

    es = [jnp.ones((tb, 1), jnp.float32)]
    for k in range(1, _TOPK):
        es.append(jnp.exp(vals[k] - vals[0]))
    total = es[0]
    for k in range(1, _TOPK):
        total = total + es[k]
    r = 1.0 / total
    wts8 = jnp.concatenate([e * r for e in es], axis=1)
    return idx8, wts8


def _router_body(x_ref, w_ref, logits_ref, idx_ref, wts_ref):
    logits = jnp.dot(x_ref[...], w_ref[...], preferred_element_type=jnp.float32)
    logits_ref[...] = logits
    idx8, wts8 = _topk_softmax(logits)
    idx_ref[...] = idx8
    wts_ref[...] = wts8


@functools.partial(jax.jit, static_argnames=())
def kernel(hidden_states, W_gate):
    b, s, h = hidden_states.shape
    n = b * s
    x = hidden_states.reshape(n, h)

    grid = (n // _TB,)
    logits, idx8, wts8 = pl.pallas_call(
        _router_body,
        grid=grid,
        in_specs=[
            pl.BlockSpec((_TB, h), lambda i: (i, 0)),
            pl.BlockSpec((h, _EXPERTS), lambda i: (0, 0)),
        ],
        out_specs=[
            pl.BlockSpec((_TB, _EXPERTS), lambda i: (i, 0)),
            pl.BlockSpec((_TB, _TOPK), lambda i: (i, 0)),
            pl.BlockSpec((_TB, _TOPK), lambda i: (i, 0)),
        ],
        out_shape=[
            jax.ShapeDtypeStruct((n, _EXPERTS), jnp.float32),
            jax.ShapeDtypeStruct((n, _TOPK), jnp.int32),
            jax.ShapeDtypeStruct((n, _TOPK), jnp.float32),
        ],
        compiler_params=pltpu.CompilerParams(
            dimension_semantics=("parallel",),
        ),
    )(x, W_gate)

    return (
        logits.reshape(b, s, _EXPERTS),
        idx8.reshape(b, s, _TOPK),
        wts8.reshape(b, s, _TOPK),
    )
